# Initial kernel scaffold; baseline (speedup 1.0000x reference)
import jax
import jax.numpy as jnp
from jax.experimental import pallas as pl


def kernel(feats, off_feats, cluster_center, point_queue):
    raise NotImplementedError("write your pallas kernel here")


# speedup vs baseline: 1.5897x; 1.5897x over previous
import jax
import jax.numpy as jnp
from jax.experimental import pallas as pl

DIM = 256
K = 64
PIXEL_SIZE = 50
K_BAN = 10
TEMP = 0.1
BASE_TEMP = 2.0
KNEG = PIXEL_SIZE - K_BAN
M = 4 * 1024
NQ = K * KNEG
TILE = 256
NT = M // TILE
SHIFT = 1.0 / TEMP
SCALE = -(TEMP / BASE_TEMP)


def _prep_kernel(feats_ref, off_ref, cc_ref, nf_ref, lab_ref):
    i = pl.program_id(0)
    f = feats_ref[...]
    nrm = jnp.sqrt(jnp.sum(f * f, axis=1, keepdims=True))
    nf_ref[...] = f / jnp.maximum(nrm, 1e-12)
    o = off_ref[...]
    la = jax.lax.dot_general(o, cc_ref[...], (((1,), (1,)), ((), ())),
                             preferred_element_type=jnp.float32)
    m = jnp.max(la, axis=1, keepdims=True)
    col = jax.lax.broadcasted_iota(jnp.int32, la.shape, 1)
    idx = jnp.min(jnp.where(la >= m, col, K), axis=1)
    lab_ref[0, pl.ds(i * TILE, TILE)] = idx.astype(jnp.float32)


def _loss_kernel(nf_ref, lab_ref, xq_ref, cc_ref, out_ref):
    i = pl.program_id(0)
    a = nf_ref[pl.ds(i * TILE, TILE), :]
    lab_row = lab_ref[:, pl.ds(i * TILE, TILE)]
    lab_r = jnp.transpose(lab_row)

    l1 = jax.lax.dot_general(a, nf_ref[...], (((1,), (1,)), ((), ())),
                             preferred_element_type=jnp.float32)
    s1 = l1 * (1.0 / TEMP) - SHIFT
    t1 = jnp.exp(s1)
    mask1 = (lab_r == lab_ref[...]).astype(jnp.float32)
    colg = jax.lax.broadcasted_iota(jnp.int32, (TILE, M), 1)
    rowg = jax.lax.broadcasted_iota(jnp.int32, (TILE, M), 0) + i * TILE
    mask1 = mask1 * (1.0 - (colg == rowg).astype(jnp.float32))
    neg1 = jnp.sum((1.0 - mask1) * t1, axis=1, keepdims=True)
    cnt1 = jnp.sum(mask1, axis=1, keepdims=True)
    sum_pl = jnp.sum(mask1 * s1, axis=1, keepdims=True)
    sum_lg = jnp.sum(mask1 * jnp.log(t1 + neg1), axis=1, keepdims=True)
    mlpp1 = (sum_pl - sum_lg) / jnp.maximum(cnt1, 1.0)
    valid = (cnt1 > 0.0).astype(jnp.float32)
    ppc_num = jnp.sum(valid * SCALE * mlpp1)
    ppc_val = jnp.sum(valid)

    l2 = jax.lax.dot_general(a, xq_ref[...], (((1,), (1,)), ((), ())),
                             preferred_element_type=jnp.float32)
    s2 = l2 * (1.0 / TEMP) - SHIFT
    t2 = jnp.exp(s2)
    colc = (jax.lax.broadcasted_iota(jnp.int32, (TILE, NQ), 1)
            // KNEG).astype(jnp.float32)
    mask2 = (lab_r == colc).astype(jnp.float32)
    neg2 = jnp.sum((1.0 - mask2) * t2, axis=1, keepdims=True)
    sum_pl2 = jnp.sum(mask2 * s2, axis=1, keepdims=True)
    sum_lg2 = jnp.sum(mask2 * jnp.log(t2 + neg2), axis=1, keepdims=True)
    ppc2_num = jnp.sum(SCALE * (sum_pl2 - sum_lg2) / float(KNEG))

    l3 = jax.lax.dot_general(a, cc_ref[...], (((1,), (1,)), ((), ())),
                             preferred_element_type=jnp.float32)
    s3 = l3 * (1.0 / TEMP) - SHIFT
    t3 = jnp.exp(s3)
    colk = jax.lax.broadcasted_iota(jnp.int32, (TILE, K), 1).astype(jnp.float32)
    mask3 = (lab_r == colk).astype(jnp.float32)
    neg3 = jnp.sum((1.0 - mask3) * t3, axis=1, keepdims=True)
    sum_pl3 = jnp.sum(mask3 * s3, axis=1, keepdims=True)
    sum_lg3 = jnp.sum(mask3 * jnp.log(t3 + neg3), axis=1, keepdims=True)
    pcc_num = jnp.sum(SCALE * (sum_pl3 - sum_lg3))

    lane = jax.lax.broadcasted_iota(jnp.int32, (1, 128), 1)
    part = (jnp.where(lane == 0, ppc_num, 0.0)
            + jnp.where(lane == 1, ppc_val, 0.0)
            + jnp.where(lane == 2, ppc2_num, 0.0)
            + jnp.where(lane == 3, pcc_num, 0.0))
    out_ref[...] = jnp.where(i == 0, part, out_ref[...] + part)


def kernel(feats, off_feats, cluster_center, point_queue):
    feats2 = feats.reshape(M, DIM)
    off2 = off_feats.reshape(M, DIM)
    xq = point_queue[:, :KNEG, :].reshape(NQ, DIM)

    nf, labels = pl.pallas_call(
        _prep_kernel,
        grid=(NT,),
        in_specs=[
            pl.BlockSpec((TILE, DIM), lambda i: (i, 0)),
            pl.BlockSpec((TILE, DIM), lambda i: (i, 0)),
            pl.BlockSpec((K, DIM), lambda i: (0, 0)),
        ],
        out_specs=[
            pl.BlockSpec((TILE, DIM), lambda i: (i, 0)),
            pl.BlockSpec((1, M), lambda i: (0, 0)),
        ],
        out_shape=[
            jax.ShapeDtypeStruct((M, DIM), jnp.float32),
            jax.ShapeDtypeStruct((1, M), jnp.float32),
        ],
    )(feats2, off2, cluster_center)

    parts = pl.pallas_call(
        _loss_kernel,
        grid=(NT,),
        in_specs=[
            pl.BlockSpec((M, DIM), lambda i: (0, 0)),
            pl.BlockSpec((1, M), lambda i: (0, 0)),
            pl.BlockSpec((NQ, DIM), lambda i: (0, 0)),
            pl.BlockSpec((K, DIM), lambda i: (0, 0)),
        ],
        out_specs=pl.BlockSpec((1, 128), lambda i: (0, 0)),
        out_shape=jax.ShapeDtypeStruct((1, 128), jnp.float32),
    )(nf, labels, xq, cluster_center)

    p = parts[0]
    loss_ppc = p[0] / jnp.maximum(p[1], 1.0)
    loss_ppc2 = p[2] / float(M)
    loss_pcc = p[3] / float(M)
    return loss_ppc + loss_ppc2 + loss_pcc


# speedup vs baseline: 1.8620x; 1.1713x over previous
import jax
import jax.numpy as jnp
from jax.experimental import pallas as pl

DIM = 256
K = 64
PIXEL_SIZE = 50
K_BAN = 10
TEMP = 0.1
BASE_TEMP = 2.0
KNEG = PIXEL_SIZE - K_BAN
M = 4 * 1024
NQ = K * KNEG
TILE = 256
NT = M // TILE
SCALE = -(TEMP / BASE_TEMP)
RSQ = 1.0 / TEMP ** 0.5


def _prep_kernel(feats_ref, off_ref, cc_ref, nf_ref, lab_ref, ohc_ref,
                 hist_ref):
    i = pl.program_id(0)
    f = feats_ref[...]
    nrm = jnp.sqrt(jnp.sum(f * f, axis=1, keepdims=True))
    nf_ref[...] = f * (RSQ / jnp.maximum(nrm, 1e-12))
    o = off_ref[...]
    la = jax.lax.dot_general(o, cc_ref[...], (((1,), (1,)), ((), ())),
                             preferred_element_type=jnp.float32)
    m = jnp.max(la, axis=1, keepdims=True)
    col = jax.lax.broadcasted_iota(jnp.int32, la.shape, 1)
    idx = jnp.min(jnp.where(la >= m, col, K), axis=1, keepdims=True)
    labf = idx.astype(jnp.float32)
    lab_ref[...] = labf
    oh = (idx == jax.lax.broadcasted_iota(jnp.int32, (TILE, K), 1))
    ohf = oh.astype(jnp.float32)
    ohc_ref[...] = ohf
    part = jnp.sum(ohf, axis=0, keepdims=True)
    hist_ref[...] = jnp.where(i == 0, part, hist_ref[...] + part)


def _loss_kernel(nf_ref, lab_ref, ohc_ref, hist_ref, xq_ref, cc_ref, b_ref,
                 out_ref):
    i = pl.program_id(0)
    a = nf_ref[pl.ds(i * TILE, TILE), :]
    lab_r = lab_ref[pl.ds(i * TILE, TILE), :]
    selc = (lab_r == jax.lax.broadcasted_iota(
        jnp.int32, (TILE, K), 1).astype(jnp.float32)).astype(jnp.float32)

    l1 = jax.lax.dot_general(a, nf_ref[...], (((1,), (1,)), ((), ())),
                             preferred_element_type=jnp.float32)
    t1 = jnp.exp(l1)
    t1b = jnp.dot(t1, ohc_ref[...], preferred_element_type=jnp.float32)
    s1b = jnp.dot(l1, ohc_ref[...], preferred_element_type=jnp.float32)
    lii = jnp.sum(a * a, axis=1, keepdims=True)
    tii = jnp.exp(lii)
    sum_t = jnp.sum(t1b, axis=1, keepdims=True)
    pos_t = jnp.sum(selc * t1b, axis=1, keepdims=True)
    neg1 = sum_t - pos_t + tii
    lg1 = jnp.log(t1 + neg1)
    lg1b = jnp.dot(lg1, ohc_ref[...], preferred_element_type=jnp.float32)
    sum_pl = jnp.sum(selc * s1b, axis=1, keepdims=True) - lii
    sum_lg = jnp.sum(selc * lg1b, axis=1, keepdims=True) - jnp.log(tii + neg1)
    cnt = jnp.sum(selc * hist_ref[...], axis=1, keepdims=True) - 1.0
    mlpp1 = (sum_pl - sum_lg) / jnp.maximum(cnt, 1.0)
    valid = (cnt > 0.0).astype(jnp.float32)
    ppc_num = jnp.sum(valid * SCALE * mlpp1)
    ppc_val = jnp.sum(valid)

    l2 = jax.lax.dot_general(a, xq_ref[...], (((1,), (1,)), ((), ())),
                             preferred_element_type=jnp.float32)
    t2 = jnp.exp(l2)
    t2b = jnp.dot(t2, b_ref[...], preferred_element_type=jnp.float32)
    s2b = jnp.dot(l2, b_ref[...], preferred_element_type=jnp.float32)
    sum_t2 = jnp.sum(t2b, axis=1, keepdims=True)
    pos_t2 = jnp.sum(selc * t2b, axis=1, keepdims=True)
    neg2 = sum_t2 - pos_t2
    lg2 = jnp.log(t2 + neg2)
    lg2b = jnp.dot(lg2, b_ref[...], preferred_element_type=jnp.float32)
    num2 = jnp.sum(selc * (s2b - lg2b), axis=1, keepdims=True)
    ppc2_num = jnp.sum(SCALE * num2 / float(KNEG))

    l3 = jax.lax.dot_general(a, cc_ref[...], (((1,), (1,)), ((), ())),
                             preferred_element_type=jnp.float32)
    t3 = jnp.exp(l3)
    sum_t3 = jnp.sum(t3, axis=1, keepdims=True)
    pos_t3 = jnp.sum(selc * t3, axis=1, keepdims=True)
    pos_l3 = jnp.sum(selc * l3, axis=1, keepdims=True)
    neg3 = sum_t3 - pos_t3
    mlpp3 = pos_l3 - jnp.log(pos_t3 + neg3)
    pcc_num = jnp.sum(SCALE * mlpp3)

    lane = jax.lax.broadcasted_iota(jnp.int32, (1, 128), 1)
    part = (jnp.where(lane == 0, ppc_num, 0.0)
            + jnp.where(lane == 1, ppc_val, 0.0)
            + jnp.where(lane == 2, ppc2_num, 0.0)
            + jnp.where(lane == 3, pcc_num, 0.0))
    out_ref[...] = jnp.where(i == 0, part, out_ref[...] + part)


def kernel(feats, off_feats, cluster_center, point_queue):
    feats2 = feats.reshape(M, DIM)
    off2 = off_feats.reshape(M, DIM)
    xq = point_queue[:, :KNEG, :].reshape(NQ, DIM) * RSQ
    ccs = cluster_center * RSQ
    bmat = (jnp.arange(NQ, dtype=jnp.int32)[:, None] // KNEG
            == jnp.arange(K, dtype=jnp.int32)[None, :]).astype(jnp.float32)

    nf, labels, ohc, hist = pl.pallas_call(
        _prep_kernel,
        grid=(NT,),
        in_specs=[
            pl.BlockSpec((TILE, DIM), lambda i: (i, 0)),
            pl.BlockSpec((TILE, DIM), lambda i: (i, 0)),
            pl.BlockSpec((K, DIM), lambda i: (0, 0)),
        ],
        out_specs=[
            pl.BlockSpec((TILE, DIM), lambda i: (i, 0)),
            pl.BlockSpec((TILE, 1), lambda i: (i, 0)),
            pl.BlockSpec((TILE, K), lambda i: (i, 0)),
            pl.BlockSpec((1, K), lambda i: (0, 0)),
        ],
        out_shape=[
            jax.ShapeDtypeStruct((M, DIM), jnp.float32),
            jax.ShapeDtypeStruct((M, 1), jnp.float32),
            jax.ShapeDtypeStruct((M, K), jnp.float32),
            jax.ShapeDtypeStruct((1, K), jnp.float32),
        ],
    )(feats2, off2, cluster_center)

    parts = pl.pallas_call(
        _loss_kernel,
        grid=(NT,),
        in_specs=[
            pl.BlockSpec((M, DIM), lambda i: (0, 0)),
            pl.BlockSpec((M, 1), lambda i: (0, 0)),
            pl.BlockSpec((M, K), lambda i: (0, 0)),
            pl.BlockSpec((1, K), lambda i: (0, 0)),
            pl.BlockSpec((NQ, DIM), lambda i: (0, 0)),
            pl.BlockSpec((K, DIM), lambda i: (0, 0)),
            pl.BlockSpec((NQ, K), lambda i: (0, 0)),
        ],
        out_specs=pl.BlockSpec((1, 128), lambda i: (0, 0)),
        out_shape=jax.ShapeDtypeStruct((1, 128), jnp.float32),
    )(nf, labels, ohc, hist, xq, ccs, bmat)

    p = parts[0]
    loss_ppc = p[0] / jnp.maximum(p[1], 1.0)
    loss_ppc2 = p[2] / float(M)
    loss_pcc = p[3] / float(M)
    return loss_ppc + loss_ppc2 + loss_pcc


# speedup vs baseline: 2.0418x; 1.0966x over previous
import jax
import jax.numpy as jnp
from jax.experimental import pallas as pl
from jax.experimental.pallas import tpu as pltpu

DIM = 256
K = 64
PIXEL_SIZE = 50
K_BAN = 10
TEMP = 0.1
BASE_TEMP = 2.0
KNEG = PIXEL_SIZE - K_BAN
M = 4 * 1024
NQ = K * KNEG
TILE = 256
NT = M // TILE
SCALE = -(TEMP / BASE_TEMP)
RSQ = 1.0 / TEMP ** 0.5


def _prep_kernel(feats_ref, off_ref, cc_ref, nf_ref, lab_ref, ohc_ref,
                 hist_ref):
    i = pl.program_id(0)
    f = feats_ref[...]
    nrm = jnp.sqrt(jnp.sum(f * f, axis=1, keepdims=True))
    nf_ref[...] = (f * (RSQ / jnp.maximum(nrm, 1e-12))).astype(jnp.bfloat16)
    o = off_ref[...]
    la = jax.lax.dot_general(o, cc_ref[...], (((1,), (1,)), ((), ())),
                             preferred_element_type=jnp.float32)
    m = jnp.max(la, axis=1, keepdims=True)
    col = jax.lax.broadcasted_iota(jnp.int32, la.shape, 1)
    idx = jnp.min(jnp.where(la >= m, col, K), axis=1, keepdims=True)
    lab_ref[...] = idx.astype(jnp.float32)
    oh = (idx == jax.lax.broadcasted_iota(jnp.int32, (TILE, K), 1))
    ohf = oh.astype(jnp.float32)
    ohc_ref[...] = ohf.astype(jnp.bfloat16)
    part = jnp.sum(ohf, axis=0, keepdims=True)
    hist_ref[...] = jnp.where(i == 0, part, hist_ref[...] + part)


def _loss_kernel(nf_ref, lab_ref, ohc_ref, hist_ref, xq_ref, cc_ref, b_ref,
                 out_ref, cs_ref, xb_ref):
    i = pl.program_id(0)

    @pl.when(i == 0)
    def _():
        cs_ref[...] = jax.lax.dot_general(
            nf_ref[...], ohc_ref[...], (((0,), (0,)), ((), ())),
            preferred_element_type=jnp.float32)
        xb_ref[...] = jax.lax.dot_general(
            xq_ref[...], b_ref[...], (((0,), (0,)), ((), ())),
            preferred_element_type=jnp.float32)

    a = nf_ref[pl.ds(i * TILE, TILE), :]
    lab_r = lab_ref[pl.ds(i * TILE, TILE), :]
    selc = (lab_r == jax.lax.broadcasted_iota(
        jnp.int32, (TILE, K), 1).astype(jnp.float32)).astype(jnp.float32)

    l1 = jax.lax.dot_general(a, nf_ref[...], (((1,), (1,)), ((), ())),
                             preferred_element_type=jnp.float32)
    t1 = jnp.exp(l1.astype(jnp.bfloat16))
    t1b = jnp.dot(t1, ohc_ref[...], preferred_element_type=jnp.float32)
    s1b = jnp.dot(a, cs_ref[...].astype(jnp.bfloat16),
                  preferred_element_type=jnp.float32)
    af = a.astype(jnp.float32)
    lii = jnp.sum(af * af, axis=1, keepdims=True)
    tii = jnp.exp(lii.astype(jnp.bfloat16).astype(jnp.float32))
    tii = tii.astype(jnp.bfloat16).astype(jnp.float32)
    sum_t = jnp.sum(t1b, axis=1, keepdims=True)
    pos_t = jnp.sum(selc * t1b, axis=1, keepdims=True)
    neg1 = sum_t - pos_t + tii
    lg1 = jnp.log(t1 + neg1.astype(jnp.bfloat16))
    lg1b = jnp.dot(lg1, ohc_ref[...], preferred_element_type=jnp.float32)
    sum_pl = jnp.sum(selc * s1b, axis=1, keepdims=True) - lii
    sum_lg = jnp.sum(selc * lg1b, axis=1, keepdims=True) - jnp.log(tii + neg1)
    cnt = jnp.sum(selc * hist_ref[...], axis=1, keepdims=True) - 1.0
    mlpp1 = (sum_pl - sum_lg) / jnp.maximum(cnt, 1.0)
    valid = (cnt > 0.0).astype(jnp.float32)
    ppc_num = jnp.sum(valid * SCALE * mlpp1)
    ppc_val = jnp.sum(valid)

    l2 = jax.lax.dot_general(a, xq_ref[...], (((1,), (1,)), ((), ())),
                             preferred_element_type=jnp.float32)
    t2 = jnp.exp(l2.astype(jnp.bfloat16))
    t2b = jnp.dot(t2, b_ref[...], preferred_element_type=jnp.float32)
    s2b = jnp.dot(a, xb_ref[...].astype(jnp.bfloat16),
                  preferred_element_type=jnp.float32)
    sum_t2 = jnp.sum(t2b, axis=1, keepdims=True)
    pos_t2 = jnp.sum(selc * t2b, axis=1, keepdims=True)
    neg2 = sum_t2 - pos_t2
    lg2 = jnp.log(t2 + neg2.astype(jnp.bfloat16))
    lg2b = jnp.dot(lg2, b_ref[...], preferred_element_type=jnp.float32)
    num2 = jnp.sum(selc * (s2b - lg2b), axis=1, keepdims=True)
    ppc2_num = jnp.sum(SCALE * num2 / float(KNEG))

    l3 = jax.lax.dot_general(a, cc_ref[...], (((1,), (1,)), ((), ())),
                             preferred_element_type=jnp.float32)
    t3 = jnp.exp(l3)
    sum_t3 = jnp.sum(t3, axis=1, keepdims=True)
    pos_t3 = jnp.sum(selc * t3, axis=1, keepdims=True)
    pos_l3 = jnp.sum(selc * l3, axis=1, keepdims=True)
    neg3 = sum_t3 - pos_t3
    mlpp3 = pos_l3 - jnp.log(pos_t3 + neg3)
    pcc_num = jnp.sum(SCALE * mlpp3)

    lane = jax.lax.broadcasted_iota(jnp.int32, (1, 128), 1)
    part = (jnp.where(lane == 0, ppc_num, 0.0)
            + jnp.where(lane == 1, ppc_val, 0.0)
            + jnp.where(lane == 2, ppc2_num, 0.0)
            + jnp.where(lane == 3, pcc_num, 0.0))
    out_ref[...] = jnp.where(i == 0, part, out_ref[...] + part)


def kernel(feats, off_feats, cluster_center, point_queue):
    feats2 = feats.reshape(M, DIM)
    off2 = off_feats.reshape(M, DIM)
    xq = (point_queue[:, :KNEG, :].reshape(NQ, DIM) * RSQ).astype(jnp.bfloat16)
    ccs = (cluster_center * RSQ).astype(jnp.bfloat16)
    bmat = (jnp.arange(NQ, dtype=jnp.int32)[:, None] // KNEG
            == jnp.arange(K, dtype=jnp.int32)[None, :]).astype(jnp.bfloat16)

    nf, labels, ohc, hist = pl.pallas_call(
        _prep_kernel,
        grid=(NT,),
        in_specs=[
            pl.BlockSpec((TILE, DIM), lambda i: (i, 0)),
            pl.BlockSpec((TILE, DIM), lambda i: (i, 0)),
            pl.BlockSpec((K, DIM), lambda i: (0, 0)),
        ],
        out_specs=[
            pl.BlockSpec((TILE, DIM), lambda i: (i, 0)),
            pl.BlockSpec((TILE, 1), lambda i: (i, 0)),
            pl.BlockSpec((TILE, K), lambda i: (i, 0)),
            pl.BlockSpec((1, K), lambda i: (0, 0)),
        ],
        out_shape=[
            jax.ShapeDtypeStruct((M, DIM), jnp.bfloat16),
            jax.ShapeDtypeStruct((M, 1), jnp.float32),
            jax.ShapeDtypeStruct((M, K), jnp.bfloat16),
            jax.ShapeDtypeStruct((1, K), jnp.float32),
        ],
    )(feats2, off2, cluster_center)

    parts = pl.pallas_call(
        _loss_kernel,
        grid=(NT,),
        in_specs=[
            pl.BlockSpec((M, DIM), lambda i: (0, 0)),
            pl.BlockSpec((M, 1), lambda i: (0, 0)),
            pl.BlockSpec((M, K), lambda i: (0, 0)),
            pl.BlockSpec((1, K), lambda i: (0, 0)),
            pl.BlockSpec((NQ, DIM), lambda i: (0, 0)),
            pl.BlockSpec((K, DIM), lambda i: (0, 0)),
            pl.BlockSpec((NQ, K), lambda i: (0, 0)),
        ],
        out_specs=pl.BlockSpec((1, 128), lambda i: (0, 0)),
        out_shape=jax.ShapeDtypeStruct((1, 128), jnp.float32),
        scratch_shapes=[
            pltpu.VMEM((DIM, K), jnp.float32),
            pltpu.VMEM((DIM, K), jnp.float32),
        ],
    )(nf, labels, ohc, hist, xq, ccs, bmat)

    p = parts[0]
    loss_ppc = p[0] / jnp.maximum(p[1], 1.0)
    loss_ppc2 = p[2] / float(M)
    loss_pcc = p[3] / float(M)
    return loss_ppc + loss_ppc2 + loss_pcc
